# baseline (speedup 1.0000x reference)
import jax
import jax.numpy as jnp
from jax.experimental import pallas as pl

_N = 512
_U = 128


def _prep_body(adj_ref, adjt_ref, x_ref, poi_ref, wpoi_ref, bp_ref,
               sd_ref, x1_ref, x2_ref, pb_ref):
    adj = adj_ref[...]
    s = adj / jnp.clip(jnp.sum(adj, axis=1, keepdims=True), 1e-8, None)
    sb = s.astype(jnp.bfloat16)
    s2 = jnp.dot(sb, sb, preferred_element_type=jnp.float32)
    sd_ref[0:_N] = sb
    sd_ref[_N:] = (2.0 * s2).astype(jnp.bfloat16)
    adjt = adjt_ref[...]
    st = adjt / jnp.clip(jnp.sum(adjt, axis=0, keepdims=True), 1e-8, None)
    x0 = x_ref[...]
    x1 = jnp.dot(x0, st)
    x2 = 2.0 * jnp.dot(x1, st) - x0
    x1_ref[...] = x1
    x2_ref[...] = x2
    pb_ref[...] = jnp.dot(poi_ref[...], wpoi_ref[...]) + bp_ref[0, 0]


def _dot(a, b):
    return jnp.dot(a, b, preferred_element_type=jnp.float32)


_BT = 8
_G = 4


def _main_body(s_ref, x0_ref, x1_ref, x2_ref,
               h0_ref, h1_ref,
               wg0h_ref, wg0x_ref, bg0_ref,
               wc0h_ref, wc0x_ref, bc0_ref,
               wg1_ref, bg1_ref, wc1_ref, bc1_ref,
               wph_ref, pb_ref,
               out_ref, ho_ref):
    sd = s_ref[...]
    bf = jnp.bfloat16
    h0l = [h0_ref[i] for i in range(_BT)]
    h1l = [h1_ref[i] for i in range(_BT)]
    xs = jnp.concatenate(
        [jnp.concatenate([x0_ref[i], x1_ref[i], x2_ref[i]], axis=1)
         for i in range(_BT)], axis=0).astype(bf)

    wg0h = wg0h_ref[...]
    wc0h = wc0h_ref[...]
    wg1 = wg1_ref[...]
    wc1 = wc1_ref[...]
    w = 2 * _U

    for base in range(0, _BT, _G):
        items = range(base, base + _G)
        h0c = jnp.concatenate([h0l[i].astype(bf) for i in items], axis=1)
        d0 = _dot(sd, h0c)
        g1c = d0[:_N].astype(bf)
        g2c = (d0[_N:] - h0c).astype(bf)
        rhl = []
        ul = []
        for j, i in enumerate(items):
            sl = slice(j*_U, (j+1)*_U)
            xsl = xs[i*_N:(i+1)*_N]
            gate = (_dot(h0c[:, sl], wg0h[:_U]) + _dot(g1c[:, sl], wg0h[_U:2*_U])
                    + _dot(g2c[:, sl], wg0h[2*_U:])
                    + xsl[:, 0:1] * wg0x_ref[0:1] + xsl[:, 1:2] * wg0x_ref[1:2]
                    + xsl[:, 2:3] * wg0x_ref[2:3])
            gate = jax.nn.sigmoid(gate + bg0_ref[...])
            ul.append(gate[:, _U:])
            rhl.append((gate[:, :_U] * h0l[i]).astype(bf))
        rhc = jnp.concatenate(rhl, axis=1)
        d1 = _dot(sd, rhc)
        c1c = d1[:_N].astype(bf)
        c2c = (d1[_N:] - rhc).astype(bf)
        hn0bl = []
        for j, i in enumerate(items):
            sl = slice(j*_U, (j+1)*_U)
            xsl = xs[i*_N:(i+1)*_N]
            cand = (_dot(rhc[:, sl], wc0h[:_U]) + _dot(c1c[:, sl], wc0h[_U:2*_U])
                    + _dot(c2c[:, sl], wc0h[2*_U:])
                    + xsl[:, 0:1] * wc0x_ref[0:1] + xsl[:, 1:2] * wc0x_ref[1:2]
                    + xsl[:, 2:3] * wc0x_ref[2:3])
            cand = jnp.tanh(cand + bc0_ref[...])
            hn0 = ul[j] * h0l[i] + (1.0 - ul[j]) * cand
            ho_ref[0, i] = hn0
            hn0bl.append(hn0.astype(bf))

        xx0c = jnp.concatenate(
            [jnp.concatenate([hn0bl[j], h1l[i].astype(bf)], axis=1)
             for j, i in enumerate(items)], axis=1)
        d2 = _dot(sd, xx0c)
        xx1c = d2[:_N].astype(bf)
        xx2c = (d2[_N:] - xx0c).astype(bf)
        rh1l = []
        u1l = []
        for j, i in enumerate(items):
            sl = slice(j*w, (j+1)*w)
            gate1 = (_dot(xx0c[:, sl], wg1[:w]) + _dot(xx1c[:, sl], wg1[w:2*w])
                     + _dot(xx2c[:, sl], wg1[2*w:]))
            gate1 = jax.nn.sigmoid(gate1 + bg1_ref[...])
            u1l.append(gate1[:, _U:])
            rh1l.append((gate1[:, :_U] * h1l[i]).astype(bf))
        rh1c = jnp.concatenate(rh1l, axis=1)
        d3 = _dot(sd, rh1c)
        y1c = d3[:_N].astype(bf)
        y2c = (d3[_N:] - rh1c).astype(bf)
        for j, i in enumerate(items):
            cand1 = (_dot(hn0bl[j], wc1[:_U]) + _dot(rh1c[:, j*_U:(j+1)*_U], wc1[_U:2*_U])
                     + _dot(xx1c[:, j*w:j*w+_U], wc1[2*_U:3*_U])
                     + _dot(y1c[:, j*_U:(j+1)*_U], wc1[3*_U:4*_U])
                     + _dot(xx2c[:, j*w:j*w+_U], wc1[4*_U:5*_U])
                     + _dot(y2c[:, j*_U:(j+1)*_U], wc1[5*_U:]))
            cand1 = jnp.tanh(cand1 + bc1_ref[...])
            hn1 = u1l[j] * h1l[i] + (1.0 - u1l[j]) * cand1
            ho_ref[1, i] = hn1
            out_ref[i] = _dot(hn1.astype(bf), wph_ref[...]) + pb_ref[...]


def kernel(inputs, adj_mx, nodevec1, nodevec2, POI_feat, labels,
           hidden_state, W_gate0, b_gate0, W_cand0, b_cand0,
           W_gate1, b_gate1, W_cand1, b_cand1, W_proj, b_proj):
    B = inputs.shape[0]
    f32 = jnp.float32

    sd, x1t, x2t, pb = pl.pallas_call(
        _prep_body,
        out_shape=[
            jax.ShapeDtypeStruct((2 * _N, _N), jnp.bfloat16),
            jax.ShapeDtypeStruct((B, _N), f32),
            jax.ShapeDtypeStruct((B, _N), f32),
            jax.ShapeDtypeStruct((_N, 1), f32),
        ],
    )(adj_mx, adj_mx.T, inputs, POI_feat, W_proj[_U:], b_proj.reshape(1, 1))

    wg0h = jnp.concatenate([W_gate0[1:129], W_gate0[130:258], W_gate0[259:387]], axis=0)
    wg0x = jnp.stack([W_gate0[0], W_gate0[129], W_gate0[258]])
    wc0h = jnp.concatenate([W_cand0[1:129], W_cand0[130:258], W_cand0[259:387]], axis=0)
    wc0x = jnp.stack([W_cand0[0], W_cand0[129], W_cand0[258]])

    const2 = lambda shape: pl.BlockSpec(shape, lambda b: (0, 0))
    step3 = lambda shape: pl.BlockSpec(shape, lambda b: (b, 0, 0))

    out, ho = pl.pallas_call(
        _main_body,
        grid=(B // _BT,),
        in_specs=[
            const2((2 * _N, _N)),
            step3((_BT, _N, 1)), step3((_BT, _N, 1)), step3((_BT, _N, 1)),
            step3((_BT, _N, _U)), step3((_BT, _N, _U)),
            const2((3 * _U, 2 * _U)), const2((3, 2 * _U)), const2((1, 2 * _U)),
            const2((3 * _U, _U)), const2((3, _U)), const2((1, _U)),
            const2((6 * _U, 2 * _U)), const2((1, 2 * _U)),
            const2((6 * _U, _U)), const2((1, _U)),
            const2((_U, 1)), const2((_N, 1)),
        ],
        out_specs=[
            step3((_BT, _N, 1)),
            pl.BlockSpec((2, _BT, _N, _U), lambda b: (0, b, 0, 0)),
        ],
        out_shape=[
            jax.ShapeDtypeStruct((B, _N, 1), f32),
            jax.ShapeDtypeStruct((2, B, _N, _U), f32),
        ],
    )(
        sd,
        inputs.reshape(B, _N, 1), x1t.reshape(B, _N, 1), x2t.reshape(B, _N, 1),
        hidden_state[0].reshape(B, _N, _U), hidden_state[1].reshape(B, _N, _U),
        wg0h.astype(jnp.bfloat16), wg0x.astype(jnp.bfloat16), b_gate0.reshape(1, 2 * _U),
        wc0h.astype(jnp.bfloat16), wc0x.astype(jnp.bfloat16), b_cand0.reshape(1, _U),
        W_gate1.astype(jnp.bfloat16), b_gate1.reshape(1, 2 * _U),
        W_cand1.astype(jnp.bfloat16), b_cand1.reshape(1, _U),
        W_proj[:_U].astype(jnp.bfloat16), pb,
    )

    out_final = out.reshape(B, _N)
    hidden = ho.reshape(2, B, _N * _U)
    return (out_final, hidden)


# speedup vs baseline: 1.6472x; 1.6472x over previous
import jax
import jax.numpy as jnp
from jax.experimental import pallas as pl

_N = 512
_U = 128


def _prep_body(adj_ref, x_ref, poi_ref, wpoi_ref, bp_ref,
               sd_ref, x1_ref, x2_ref, pb_ref):
    adj = adj_ref[...]
    s = adj / jnp.clip(jnp.sum(adj, axis=1, keepdims=True), 1e-8, None)
    sb = s.astype(jnp.bfloat16)
    s2 = jnp.dot(sb, sb, preferred_element_type=jnp.float32)
    sd_ref[0:_N] = sb
    sd_ref[_N:] = (2.0 * s2).astype(jnp.bfloat16)
    st = jnp.transpose(s)
    x0 = x_ref[...]
    x1 = jnp.dot(x0, st)
    x2 = 2.0 * jnp.dot(x1, st) - x0
    x1_ref[...] = x1
    x2_ref[...] = x2
    pb = jnp.dot(poi_ref[...], wpoi_ref[...]) + bp_ref[0, 0]
    pb_ref[...] = jnp.transpose(pb)


def _dot(a, b):
    return jnp.dot(a, b, preferred_element_type=jnp.float32)


_BT = 8
_G = 4


def _main_body(s_ref, x0_ref, x1_ref, x2_ref,
               h0_ref, h1_ref,
               wg0h_ref, wg0x_ref, bg0_ref,
               wc0h_ref, wc0x_ref, bc0_ref,
               wg1_ref, bg1_ref, wc1_ref, bc1_ref,
               wph_ref, pb_ref,
               out_ref, ho_ref):
    sd = s_ref[...]
    bf = jnp.bfloat16
    h0l = [h0_ref[0, i].reshape(_N, _U) for i in range(_BT)]
    h1l = [h1_ref[0, i].reshape(_N, _U) for i in range(_BT)]
    xs = jnp.concatenate(
        [jnp.transpose(jnp.concatenate(
            [x0_ref[i:i+1], x1_ref[i:i+1], x2_ref[i:i+1]], axis=0))
         for i in range(_BT)], axis=0).astype(bf)

    wg0h = wg0h_ref[...]
    wc0h = wc0h_ref[...]
    wg1 = wg1_ref[...]
    wc1 = wc1_ref[...]
    w = 2 * _U

    for base in range(0, _BT, _G):
        items = range(base, base + _G)
        h0c = jnp.concatenate([h0l[i].astype(bf) for i in items], axis=1)
        d0 = _dot(sd, h0c)
        g1c = d0[:_N].astype(bf)
        g2c = (d0[_N:] - h0c).astype(bf)
        rhl = []
        ul = []
        for j, i in enumerate(items):
            sl = slice(j*_U, (j+1)*_U)
            xsl = xs[i*_N:(i+1)*_N]
            gate = (_dot(h0c[:, sl], wg0h[:_U]) + _dot(g1c[:, sl], wg0h[_U:2*_U])
                    + _dot(g2c[:, sl], wg0h[2*_U:])
                    + xsl[:, 0:1] * wg0x_ref[0:1] + xsl[:, 1:2] * wg0x_ref[1:2]
                    + xsl[:, 2:3] * wg0x_ref[2:3])
            gate = jax.nn.sigmoid(gate + bg0_ref[...])
            ul.append(gate[:, _U:])
            rhl.append((gate[:, :_U] * h0l[i]).astype(bf))
        rhc = jnp.concatenate(rhl, axis=1)
        d1 = _dot(sd, rhc)
        c1c = d1[:_N].astype(bf)
        c2c = (d1[_N:] - rhc).astype(bf)
        hn0bl = []
        for j, i in enumerate(items):
            sl = slice(j*_U, (j+1)*_U)
            xsl = xs[i*_N:(i+1)*_N]
            cand = (_dot(rhc[:, sl], wc0h[:_U]) + _dot(c1c[:, sl], wc0h[_U:2*_U])
                    + _dot(c2c[:, sl], wc0h[2*_U:])
                    + xsl[:, 0:1] * wc0x_ref[0:1] + xsl[:, 1:2] * wc0x_ref[1:2]
                    + xsl[:, 2:3] * wc0x_ref[2:3])
            cand = jnp.tanh(cand + bc0_ref[...])
            hn0 = ul[j] * h0l[i] + (1.0 - ul[j]) * cand
            ho_ref[0, i] = hn0.reshape(_N * _U)
            hn0bl.append(hn0.astype(bf))

        xx0c = jnp.concatenate(
            [jnp.concatenate([hn0bl[j], h1l[i].astype(bf)], axis=1)
             for j, i in enumerate(items)], axis=1)
        d2 = _dot(sd, xx0c)
        xx1c = d2[:_N].astype(bf)
        xx2c = (d2[_N:] - xx0c).astype(bf)
        rh1l = []
        u1l = []
        for j, i in enumerate(items):
            sl = slice(j*w, (j+1)*w)
            gate1 = (_dot(xx0c[:, sl], wg1[:w]) + _dot(xx1c[:, sl], wg1[w:2*w])
                     + _dot(xx2c[:, sl], wg1[2*w:]))
            gate1 = jax.nn.sigmoid(gate1 + bg1_ref[...])
            u1l.append(gate1[:, _U:])
            rh1l.append((gate1[:, :_U] * h1l[i]).astype(bf))
        rh1c = jnp.concatenate(rh1l, axis=1)
        d3 = _dot(sd, rh1c)
        y1c = d3[:_N].astype(bf)
        y2c = (d3[_N:] - rh1c).astype(bf)
        for j, i in enumerate(items):
            cand1 = (_dot(hn0bl[j], wc1[:_U]) + _dot(rh1c[:, j*_U:(j+1)*_U], wc1[_U:2*_U])
                     + _dot(xx1c[:, j*w:j*w+_U], wc1[2*_U:3*_U])
                     + _dot(y1c[:, j*_U:(j+1)*_U], wc1[3*_U:4*_U])
                     + _dot(xx2c[:, j*w:j*w+_U], wc1[4*_U:5*_U])
                     + _dot(y2c[:, j*_U:(j+1)*_U], wc1[5*_U:]))
            cand1 = jnp.tanh(cand1 + bc1_ref[...])
            hn1 = u1l[j] * h1l[i] + (1.0 - u1l[j]) * cand1
            ho_ref[1, i] = hn1.reshape(_N * _U)
            proj = _dot(hn1.astype(bf), wph_ref[...])
            out_ref[i] = jnp.transpose(proj)[0] + pb_ref[0]


def kernel(inputs, adj_mx, nodevec1, nodevec2, POI_feat, labels,
           hidden_state, W_gate0, b_gate0, W_cand0, b_cand0,
           W_gate1, b_gate1, W_cand1, b_cand1, W_proj, b_proj):
    B = inputs.shape[0]
    f32 = jnp.float32

    sd, x1t, x2t, pb = pl.pallas_call(
        _prep_body,
        out_shape=[
            jax.ShapeDtypeStruct((2 * _N, _N), jnp.bfloat16),
            jax.ShapeDtypeStruct((B, _N), f32),
            jax.ShapeDtypeStruct((B, _N), f32),
            jax.ShapeDtypeStruct((1, _N), f32),
        ],
    )(adj_mx, inputs, POI_feat, W_proj[_U:], b_proj.reshape(1, 1))

    wg0h = jnp.concatenate([W_gate0[1:129], W_gate0[130:258], W_gate0[259:387]], axis=0)
    wg0x = jnp.stack([W_gate0[0], W_gate0[129], W_gate0[258]])
    wc0h = jnp.concatenate([W_cand0[1:129], W_cand0[130:258], W_cand0[259:387]], axis=0)
    wc0x = jnp.stack([W_cand0[0], W_cand0[129], W_cand0[258]])

    const2 = lambda shape: pl.BlockSpec(shape, lambda b: (0, 0))
    step2 = lambda shape: pl.BlockSpec(shape, lambda b: (b, 0))

    out, ho = pl.pallas_call(
        _main_body,
        grid=(B // _BT,),
        in_specs=[
            const2((2 * _N, _N)),
            step2((_BT, _N)), step2((_BT, _N)), step2((_BT, _N)),
            pl.BlockSpec((1, _BT, _N * _U), lambda b: (0, b, 0)),
            pl.BlockSpec((1, _BT, _N * _U), lambda b: (1, b, 0)),
            const2((3 * _U, 2 * _U)), const2((3, 2 * _U)), const2((1, 2 * _U)),
            const2((3 * _U, _U)), const2((3, _U)), const2((1, _U)),
            const2((6 * _U, 2 * _U)), const2((1, 2 * _U)),
            const2((6 * _U, _U)), const2((1, _U)),
            const2((_U, 1)), const2((1, _N)),
        ],
        out_specs=[
            pl.BlockSpec((_BT, _N), lambda b: (b, 0)),
            pl.BlockSpec((2, _BT, _N * _U), lambda b: (0, b, 0)),
        ],
        out_shape=[
            jax.ShapeDtypeStruct((B, _N), f32),
            jax.ShapeDtypeStruct((2, B, _N * _U), f32),
        ],
    )(
        sd,
        inputs, x1t, x2t,
        hidden_state, hidden_state,
        wg0h.astype(jnp.bfloat16), wg0x.astype(jnp.bfloat16), b_gate0.reshape(1, 2 * _U),
        wc0h.astype(jnp.bfloat16), wc0x.astype(jnp.bfloat16), b_cand0.reshape(1, _U),
        W_gate1.astype(jnp.bfloat16), b_gate1.reshape(1, 2 * _U),
        W_cand1.astype(jnp.bfloat16), b_cand1.reshape(1, _U),
        W_proj[:_U].astype(jnp.bfloat16), pb,
    )

    return (out, ho)
